# baseline (speedup 1.0000x reference)
import functools

import jax
import jax.numpy as jnp
from jax import lax
from jax.experimental import pallas as pl
from jax.experimental.pallas import tpu as pltpu
from jax.experimental.pallas import tpu_sc as plsc

_H, _W = 3072, 4096
_LUT_SIZE = 4096
_L = 16

_info = plsc.get_sparse_core_info()
_NC, _NS = _info.num_cores, _info.num_subcores
_NW = _NC * _NS
_ROWS_PER_W = _H // _NW
_CR = 8
_NCHUNK = _ROWS_PER_W // _CR
_VECS = _CR * _W // _L


@functools.partial(
    pl.kernel,
    mesh=plsc.VectorSubcoreMesh(core_axis_name="c", subcore_axis_name="s"),
    out_type=jax.ShapeDtypeStruct((8, _W), jnp.float32),
    scratch_types=[
        pltpu.VMEM((_LUT_SIZE,), jnp.float32),
        pltpu.VMEM((_CR, _W), jnp.float32),
        pltpu.VMEM((_CR, _W), jnp.float32),
        pltpu.VMEM((_CR, _W), jnp.float32),
        pltpu.SemaphoreType.DMA,
        pltpu.SemaphoreType.DMA,
        pltpu.SemaphoreType.DMA,
    ],
    compiler_params=pltpu.CompilerParams(
        needs_layout_passes=False, skip_device_barrier=True),
)
def _decompand_sc(x_hbm, lut_hbm, out_hbm, lut_v, b0, b1, b2, s0, s1, s2):
    wid = lax.axis_index("s") * _NC + lax.axis_index("c")
    base = wid * _ROWS_PER_W
    bufs = (b0, b1, b2)
    sems = (s0, s1, s2)

    pltpu.sync_copy(lut_hbm, lut_v)

    @plsc.parallel_loop(0, _LUT_SIZE // _L, unroll=8)
    def _(i):
        v = lut_v[pl.ds(i * _L, _L)]
        lut_v[pl.ds(i * _L, _L)] = jnp.minimum(jnp.maximum(v, 0.0), 1.0)

    def start_in(c, b, sem):
        pltpu.async_copy(x_hbm.at[pl.ds(base + c * _CR, _CR)], b, sem)

    def start_out(c, b, sem):
        pltpu.async_copy(b, out_hbm.at[pl.ds(0, _CR)], sem)

    def wait(b, sem):
        pltpu.make_async_copy(x_hbm.at[pl.ds(base, _CR)], b, sem).wait()

    def compute(xb):
        @plsc.parallel_loop(0, _VECS, unroll=8)
        def _(i):
            r = i >> 8
            col = (i & 255) * _L
            bits = plsc.bitcast(xb[r, pl.ds(col, _L)], jnp.int32)
            idx = jnp.minimum(jnp.maximum(bits, 0), _LUT_SIZE - 1)
            xb[r, pl.ds(col, _L)] = plsc.load_gather(lut_v, [idx])

    start_in(0, bufs[0], sems[0])
    start_in(1, bufs[1], sems[1])
    for c in range(_NCHUNK):
        b, sem = bufs[c % 3], sems[c % 3]
        wait(b, sem)
        compute(b)
        start_out(c, b, sem)
        if c + 2 < _NCHUNK:
            b2, sem2 = bufs[(c - 1) % 3], sems[(c - 1) % 3]
            if c > 0:
                wait(b2, sem2)
            start_in(c + 2, b2, sem2)
    for j in range(3):
        wait(bufs[j], sems[j])


@jax.jit
def kernel(x, lut):
    xf = lax.bitcast_convert_type(x, jnp.float32)
    return _decompand_sc(xf, lut)


# speedup vs baseline: 1.4740x; 1.4740x over previous
import functools

import jax
import jax.numpy as jnp
from jax import lax
from jax.experimental import pallas as pl
from jax.experimental.pallas import tpu as pltpu
from jax.experimental.pallas import tpu_sc as plsc

_H, _W = 3072, 4096
_LUT_SIZE = 4096
_L = 16

_info = plsc.get_sparse_core_info()
_NC, _NS = _info.num_cores, _info.num_subcores
_NW = _NC * _NS
_ROWS_PER_W = _H // _NW
_CR = 8
_NCHUNK = _ROWS_PER_W // _CR
_VECS = _CR * _W // _L


@functools.partial(
    pl.kernel,
    mesh=plsc.VectorSubcoreMesh(core_axis_name="c", subcore_axis_name="s"),
    out_type=jax.ShapeDtypeStruct((_H, _W), jnp.float32),
    scratch_types=[
        pltpu.VMEM((_LUT_SIZE,), jnp.float32),
        pltpu.VMEM((_CR, _W), jnp.float32),
        pltpu.VMEM((_CR, _W), jnp.float32),
        pltpu.VMEM((_CR, _W), jnp.float32),
        pltpu.SemaphoreType.DMA,
        pltpu.SemaphoreType.DMA,
        pltpu.SemaphoreType.DMA,
    ],
    compiler_params=pltpu.CompilerParams(needs_layout_passes=False),
)
def _decompand_sc(x_hbm, lut_hbm, out_hbm, lut_v, b0, b1, b2, s0, s1, s2):
    wid = lax.axis_index("s") * _NC + lax.axis_index("c")
    base = wid * _ROWS_PER_W
    bufs = (b0, b1, b2)
    sems = (s0, s1, s2)

    pltpu.sync_copy(lut_hbm, lut_v)

    @plsc.parallel_loop(0, _LUT_SIZE // _L, unroll=8)
    def _(i):
        v = lut_v[pl.ds(i * _L, _L)]
        lut_v[pl.ds(i * _L, _L)] = jnp.minimum(jnp.maximum(v, 0.0), 1.0)

    def start_in(c, b, sem):
        pltpu.async_copy(x_hbm.at[pl.ds(base + c * _CR, _CR)], b, sem)

    def start_out(c, b, sem):
        pltpu.async_copy(b, out_hbm.at[pl.ds(base + c * _CR, _CR)], sem)

    def wait(b, sem):
        pltpu.make_async_copy(x_hbm.at[pl.ds(base, _CR)], b, sem).wait()

    def compute(xb):
        @plsc.parallel_loop(0, _VECS, unroll=8)
        def _(i):
            r = i >> 8
            col = (i & 255) * _L
            bits = plsc.bitcast(xb[r, pl.ds(col, _L)], jnp.int32)
            idx = jnp.minimum(jnp.maximum(bits, 0), _LUT_SIZE - 1)
            xb[r, pl.ds(col, _L)] = plsc.load_gather(lut_v, [idx])

    start_in(0, bufs[0], sems[0])
    start_in(1, bufs[1], sems[1])
    for c in range(_NCHUNK):
        b, sem = bufs[c % 3], sems[c % 3]
        wait(b, sem)
        compute(b)
        start_out(c, b, sem)
        if c + 2 < _NCHUNK:
            b2, sem2 = bufs[(c - 1) % 3], sems[(c - 1) % 3]
            if c > 0:
                wait(b2, sem2)
            start_in(c + 2, b2, sem2)
    for j in range(3):
        wait(bufs[j], sems[j])


@jax.jit
def kernel(x, lut):
    xf = lax.bitcast_convert_type(x, jnp.float32)
    return _decompand_sc(xf, lut)
